# baseline (speedup 1.0000x reference)
import functools

import jax
import jax.numpy as jnp
from jax import lax
from jax.experimental import pallas as pl
from jax.experimental.pallas import tpu as pltpu
from jax.experimental.pallas import tpu_sc as plsc

N = 256
FEAT = 128
LATENT = 256
VOCA = (33, 2, 2)
NTC = len(VOCA)
NSC = 2
BI = 8
GRID = N // BI

SC_NC = 2
SC_NS = 16
NW = SC_NC * SC_NS
BPW = N // NW

_DN = (((1,), (1,)), ((), ()))


@functools.partial(
    pl.kernel,
    mesh=plsc.VectorSubcoreMesh(core_axis_name="c", subcore_axis_name="s"),
    out_type=jax.ShapeDtypeStruct((NSC, N, FEAT), jnp.float32),
    scratch_types=(
        [pltpu.VMEM((BPW,), jnp.int32)] * NSC
        + [pltpu.VMEM((BPW, FEAT), jnp.float32)] * NSC
        + [pltpu.SemaphoreType.DMA]
    ),
)
def _sc_gather(t0, i0, t1, i1, out_hbm, x0, x1, r0, r1, sem):
    wid = lax.axis_index("s") * SC_NC + lax.axis_index("c")
    base = wid * BPW
    tabs = (t0, t1)
    ids = (i0, i1)
    idx = (x0, x1)
    rows = (r0, r1)
    for c in [pltpu.async_copy(ids[f].at[pl.ds(base, BPW)], idx[f], sem)
              for f in range(NSC)]:
        c.wait()
    for c in [pltpu.async_copy(tabs[f].at[idx[f]], rows[f], sem)
              for f in range(NSC)]:
        c.wait()
    for c in [pltpu.async_copy(rows[f], out_hbm.at[f, pl.ds(base, BPW)], sem)
              for f in range(NSC)]:
        c.wait()


def _pair_block(E, e_blk, W3, p1_blk, p2b):
    R = e_blk[:, None, :] * E[None, :, :]
    M = lax.dot_general(R, W3, (((2,), (1,)), ((), ())),
                        preferred_element_type=jnp.float32)
    out3 = jnp.maximum(M + p1_blk[:, None, :] + p2b[None, :, :], 0.0)
    return out3.reshape(BI * N, LATENT)


def _p_matrices(E, W, b, p1_out, p2_out):
    p1_out[...] = lax.dot_general(E, W[:, :FEAT], _DN,
                                  preferred_element_type=jnp.float32)
    p2_out[...] = (lax.dot_general(E, W[:, FEAT:2 * FEAT], _DN,
                                   preferred_element_type=jnp.float32) + b)


def _tc_onehot_body(id0, id1, id2, tab0, tab1, tab2, W_ref, b_ref,
                    o0, o1, o2, e_ref, p1_ref, p2_ref):
    ib = pl.program_id(0)

    @pl.when(ib == 0)
    def _():
        for f, (ids_ref, tab_ref) in enumerate(
                ((id0, tab0), (id1, tab1), (id2, tab2))):
            onehot = (ids_ref[...] ==
                      lax.broadcasted_iota(jnp.int32, (N, VOCA[f]), 1)
                      ).astype(jnp.float32)
            E = lax.dot_general(onehot, tab_ref[...], (((1,), (0,)), ((), ())),
                                precision=lax.Precision.HIGHEST,
                                preferred_element_type=jnp.float32)
            e_ref[f] = E
            _p_matrices(E, W_ref[f], b_ref[f], p1_ref.at[f], p2_ref.at[f])

    start = ib * BI
    outs = (o0, o1, o2)
    for f in range(NTC):
        outs[f][...] = _pair_block(e_ref[f], e_ref[f, pl.ds(start, BI), :],
                                   W_ref[f][:, 2 * FEAT:],
                                   p1_ref[f, pl.ds(start, BI), :], p2_ref[f])


def _tc_sc_body(embs_ref, W_ref, b_ref, o0, o1, p1_ref, p2_ref):
    ib = pl.program_id(0)

    @pl.when(ib == 0)
    def _():
        for f in range(NSC):
            _p_matrices(embs_ref[f], W_ref[f], b_ref[f],
                        p1_ref.at[f], p2_ref.at[f])

    start = ib * BI
    outs = (o0, o1)
    for f in range(NSC):
        outs[f][...] = _pair_block(embs_ref[f], embs_ref[f, pl.ds(start, BI), :],
                                   W_ref[f][:, 2 * FEAT:],
                                   p1_ref[f, pl.ds(start, BI), :], p2_ref[f])


@jax.jit
def _encode_onehot(id0, id1, id2, tab0, tab1, tab2, W, b):
    return pl.pallas_call(
        _tc_onehot_body,
        grid=(GRID,),
        in_specs=[pl.BlockSpec((N, 1), lambda i: (0, 0))] * NTC + [
            pl.BlockSpec((VOCA[0], FEAT), lambda i: (0, 0)),
            pl.BlockSpec((VOCA[1], FEAT), lambda i: (0, 0)),
            pl.BlockSpec((VOCA[2], FEAT), lambda i: (0, 0)),
            pl.BlockSpec((NTC, LATENT, 3 * FEAT), lambda i: (0, 0, 0)),
            pl.BlockSpec((NTC, 1, LATENT), lambda i: (0, 0, 0)),
        ],
        out_specs=[pl.BlockSpec((BI * N, LATENT), lambda i: (i, 0))] * NTC,
        out_shape=[jax.ShapeDtypeStruct((N * N, LATENT), jnp.float32)] * NTC,
        scratch_shapes=[pltpu.VMEM((NTC, N, FEAT), jnp.float32),
                        pltpu.VMEM((NTC, N, LATENT), jnp.float32),
                        pltpu.VMEM((NTC, N, LATENT), jnp.float32)],
    )(id0, id1, id2, tab0, tab1, tab2, W, b)


@jax.jit
def _encode_sc(embs, W, b):
    return pl.pallas_call(
        _tc_sc_body,
        grid=(GRID,),
        in_specs=[
            pl.BlockSpec((NSC, N, FEAT), lambda i: (0, 0, 0)),
            pl.BlockSpec((NSC, LATENT, 3 * FEAT), lambda i: (0, 0, 0)),
            pl.BlockSpec((NSC, 1, LATENT), lambda i: (0, 0, 0)),
        ],
        out_specs=[pl.BlockSpec((BI * N, LATENT), lambda i: (i, 0))] * NSC,
        out_shape=[jax.ShapeDtypeStruct((N * N, LATENT), jnp.float32)] * NSC,
        scratch_shapes=[pltpu.VMEM((NSC, N, LATENT), jnp.float32)] * 2,
    )(embs, W, b)


def kernel(typ_ids, typ_table, typ_W, typ_b, pol_ids, pol_table, pol_W, pol_b,
           mod_ids, mod_table, mod_W, mod_b, gen_ids, gen_table, gen_W, gen_b,
           ten_ids, ten_table, ten_W, ten_b):
    gids = (gen_ids.astype(jnp.int32), ten_ids.astype(jnp.int32))
    embs = _sc_gather(gen_table, gids[0], ten_table, gids[1])
    out_tc = _encode_onehot(
        typ_ids.astype(jnp.int32).reshape(N, 1),
        pol_ids.astype(jnp.int32).reshape(N, 1),
        mod_ids.astype(jnp.int32).reshape(N, 1),
        typ_table, pol_table, mod_table,
        jnp.stack((typ_W, pol_W, mod_W)),
        jnp.stack((typ_b, pol_b, mod_b)).reshape(NTC, 1, LATENT))
    out_sc = _encode_sc(embs, jnp.stack((gen_W, ten_W)),
                        jnp.stack((gen_b, ten_b)).reshape(NSC, 1, LATENT))
    return tuple(out_tc) + tuple(out_sc)


# speedup vs baseline: 1.0265x; 1.0265x over previous
import functools

import jax
import jax.numpy as jnp
from jax import lax
from jax.experimental import pallas as pl
from jax.experimental.pallas import tpu as pltpu
from jax.experimental.pallas import tpu_sc as plsc

N = 256
FEAT = 128
LATENT = 256
TVOC = 33
NSC = 4
BI = 8
GRID = N // BI

SC_NC = 2
SC_NS = 16
NW = SC_NC * SC_NS
BPW = N // NW

_DN = (((1,), (1,)), ((), ()))


@functools.partial(
    pl.kernel,
    mesh=plsc.VectorSubcoreMesh(core_axis_name="c", subcore_axis_name="s"),
    out_type=jax.ShapeDtypeStruct((NSC, N, FEAT), jnp.float32),
    scratch_types=(
        [pltpu.VMEM((BPW,), jnp.int32)] * NSC
        + [pltpu.VMEM((BPW, FEAT), jnp.float32)] * NSC
        + [pltpu.SemaphoreType.DMA]
    ),
)
def _sc_gather(t0, i0, t1, i1, t2, i2, t3, i3, out_hbm,
               x0, x1, x2, x3, r0, r1, r2, r3, sem):
    wid = lax.axis_index("s") * SC_NC + lax.axis_index("c")
    base = wid * BPW
    tabs = (t0, t1, t2, t3)
    ids = (i0, i1, i2, i3)
    idx = (x0, x1, x2, x3)
    rows = (r0, r1, r2, r3)
    for c in [pltpu.async_copy(ids[f].at[pl.ds(base, BPW)], idx[f], sem)
              for f in range(NSC)]:
        c.wait()
    for c in [pltpu.async_copy(tabs[f].at[idx[f]], rows[f], sem)
              for f in range(NSC)]:
        c.wait()
    for c in [pltpu.async_copy(rows[f], out_hbm.at[f, pl.ds(base, BPW)], sem)
              for f in range(NSC)]:
        c.wait()


def _pair_block(E, e_blk, W3, p1_blk, p2b):
    R = e_blk[:, None, :] * E[None, :, :]
    M = lax.dot_general(R, W3, (((2,), (1,)), ((), ())),
                        preferred_element_type=jnp.float32)
    out3 = jnp.maximum(M + p1_blk[:, None, :] + p2b[None, :, :], 0.0)
    return out3.reshape(BI * N, LATENT)


def _tc_typ_body(ids_ref, tab_ref, W_ref, b_ref, out_ref, e_ref, p1_ref, p2_ref):
    ib = pl.program_id(0)

    @pl.when(ib == 0)
    def _():
        onehot = (ids_ref[...] == lax.broadcasted_iota(jnp.int32, (N, TVOC), 1)
                  ).astype(jnp.float32)
        E = lax.dot_general(onehot, tab_ref[...], (((1,), (0,)), ((), ())),
                            precision=lax.Precision.HIGHEST,
                            preferred_element_type=jnp.float32)
        e_ref[...] = E
        W = W_ref[...]
        p1_ref[...] = lax.dot_general(E, W[:, :FEAT], _DN,
                                      preferred_element_type=jnp.float32)
        p2_ref[...] = (lax.dot_general(E, W[:, FEAT:2 * FEAT], _DN,
                                       preferred_element_type=jnp.float32)
                       + b_ref[...])

    start = ib * BI
    out_ref[...] = _pair_block(e_ref[...], e_ref[pl.ds(start, BI), :],
                               W_ref[..., 2 * FEAT:],
                               p1_ref[pl.ds(start, BI), :], p2_ref[...])


def _tc_rest_body(embs_ref, W0, W1, W2, W3r, b0, b1, b2, b3,
                  o0, o1, o2, o3, p1_ref, p2_ref):
    ib = pl.program_id(0)
    Ws = (W0, W1, W2, W3r)
    bs = (b0, b1, b2, b3)

    @pl.when(ib == 0)
    def _():
        for f in range(NSC):
            E = embs_ref[f]
            W = Ws[f][...]
            p1_ref[f] = lax.dot_general(E, W[:, :FEAT], _DN,
                                        preferred_element_type=jnp.float32)
            p2_ref[f] = (lax.dot_general(E, W[:, FEAT:2 * FEAT], _DN,
                                         preferred_element_type=jnp.float32)
                         + bs[f][...])

    start = ib * BI
    outs = (o0, o1, o2, o3)
    for f in range(NSC):
        outs[f][...] = _pair_block(embs_ref[f], embs_ref[f, pl.ds(start, BI), :],
                                   Ws[f][..., 2 * FEAT:],
                                   p1_ref[f, pl.ds(start, BI), :], p2_ref[f])


@jax.jit
def _encode_typ(ids, tab, W, b):
    return pl.pallas_call(
        _tc_typ_body,
        grid=(GRID,),
        in_specs=[
            pl.BlockSpec((N, 1), lambda i: (0, 0)),
            pl.BlockSpec((TVOC, FEAT), lambda i: (0, 0)),
            pl.BlockSpec((LATENT, 3 * FEAT), lambda i: (0, 0)),
            pl.BlockSpec((1, LATENT), lambda i: (0, 0)),
        ],
        out_specs=pl.BlockSpec((BI * N, LATENT), lambda i: (i, 0)),
        out_shape=jax.ShapeDtypeStruct((N * N, LATENT), jnp.float32),
        scratch_shapes=[pltpu.VMEM((N, FEAT), jnp.float32),
                        pltpu.VMEM((N, LATENT), jnp.float32),
                        pltpu.VMEM((N, LATENT), jnp.float32)],
    )(ids, tab, W, b)


@jax.jit
def _encode_rest(embs, W0, W1, W2, W3r, b0, b1, b2, b3):
    return pl.pallas_call(
        _tc_rest_body,
        grid=(GRID,),
        in_specs=(
            [pl.BlockSpec((NSC, N, FEAT), lambda i: (0, 0, 0))]
            + [pl.BlockSpec((LATENT, 3 * FEAT), lambda i: (0, 0))] * NSC
            + [pl.BlockSpec((1, LATENT), lambda i: (0, 0))] * NSC
        ),
        out_specs=[pl.BlockSpec((BI * N, LATENT), lambda i: (i, 0))] * NSC,
        out_shape=[jax.ShapeDtypeStruct((N * N, LATENT), jnp.float32)] * NSC,
        scratch_shapes=[pltpu.VMEM((NSC, N, LATENT), jnp.float32)] * 2,
    )(embs, W0, W1, W2, W3r, b0, b1, b2, b3)


def kernel(typ_ids, typ_table, typ_W, typ_b, pol_ids, pol_table, pol_W, pol_b,
           mod_ids, mod_table, mod_W, mod_b, gen_ids, gen_table, gen_W, gen_b,
           ten_ids, ten_table, ten_W, ten_b):
    ids = tuple(i.astype(jnp.int32)
                for i in (pol_ids, mod_ids, gen_ids, ten_ids))
    embs = _sc_gather(pol_table, ids[0], mod_table, ids[1],
                      gen_table, ids[2], ten_table, ids[3])
    out0 = _encode_typ(typ_ids.astype(jnp.int32).reshape(N, 1), typ_table,
                       typ_W, typ_b.reshape(1, LATENT))
    rest = _encode_rest(embs, pol_W, mod_W, gen_W, ten_W,
                        pol_b.reshape(1, LATENT), mod_b.reshape(1, LATENT),
                        gen_b.reshape(1, LATENT), ten_b.reshape(1, LATENT))
    return (out0,) + tuple(rest)


# speedup vs baseline: 1.0441x; 1.0171x over previous
import functools

import jax
import jax.numpy as jnp
from jax import lax
from jax.experimental import pallas as pl
from jax.experimental.pallas import tpu as pltpu
from jax.experimental.pallas import tpu_sc as plsc

N = 256
FEAT = 128
LATENT = 256
TVOC = 33
NSC = 4
BI = 8
GRID = N // BI

SC_NC = 2
SC_NS = 16
NW = SC_NC * SC_NS
BPW = N // NW

_DN = (((1,), (1,)), ((), ()))


@functools.partial(
    pl.kernel,
    mesh=plsc.VectorSubcoreMesh(core_axis_name="c", subcore_axis_name="s"),
    out_type=jax.ShapeDtypeStruct((NSC, N, FEAT), jnp.float32),
    scratch_types=(
        [pltpu.VMEM((BPW,), jnp.int32)] * NSC
        + [pltpu.VMEM((BPW, FEAT), jnp.float32)] * NSC
        + [pltpu.SemaphoreType.DMA]
    ),
)
def _sc_gather(t0, i0, t1, i1, t2, i2, t3, i3, out_hbm,
               x0, x1, x2, x3, r0, r1, r2, r3, sem):
    wid = lax.axis_index("s") * SC_NC + lax.axis_index("c")
    base = wid * BPW
    tabs = (t0, t1, t2, t3)
    ids = (i0, i1, i2, i3)
    idx = (x0, x1, x2, x3)
    rows = (r0, r1, r2, r3)
    for c in [pltpu.async_copy(ids[f].at[pl.ds(base, BPW)], idx[f], sem)
              for f in range(NSC)]:
        c.wait()
    for c in [pltpu.async_copy(tabs[f].at[idx[f]], rows[f], sem)
              for f in range(NSC)]:
        c.wait()
    for c in [pltpu.async_copy(rows[f], out_hbm.at[f, pl.ds(base, BPW)], sem)
              for f in range(NSC)]:
        c.wait()


def _pair_block(E, e_blk, W3, p1_blk, p2b):
    R = e_blk[:, None, :] * E[None, :, :]
    M = lax.dot_general(R, W3, (((2,), (1,)), ((), ())),
                        preferred_element_type=jnp.float32)
    out3 = jnp.maximum(M + p1_blk[:, None, :] + p2b[None, :, :], 0.0)
    return out3.reshape(BI * N, LATENT)


def _p_pair(E, W, b):
    p1 = lax.dot_general(E, W[:, :FEAT], _DN,
                         preferred_element_type=jnp.float32)
    p2 = (lax.dot_general(E, W[:, FEAT:2 * FEAT], _DN,
                          preferred_element_type=jnp.float32) + b)
    return p1, p2


def _tc_body(ids_ref, tab_ref, embs_ref, W0, W1, W2, W3r, W4,
             b0, b1, b2, b3, b4, o0, o1, o2, o3, o4,
             e_ref, p1_ref, p2_ref):
    ib = pl.program_id(0)
    Ws = (W0, W1, W2, W3r, W4)
    bs = (b0, b1, b2, b3, b4)

    @pl.when(ib == 0)
    def _():
        onehot = (ids_ref[...] == lax.broadcasted_iota(jnp.int32, (N, TVOC), 1)
                  ).astype(jnp.float32)
        e_ref[...] = lax.dot_general(
            onehot, tab_ref[...], (((1,), (0,)), ((), ())),
            precision=lax.Precision.HIGHEST,
            preferred_element_type=jnp.float32)
        for f in range(1 + NSC):
            E = e_ref[...] if f == 0 else embs_ref[f - 1]
            p1, p2 = _p_pair(E, Ws[f][...], bs[f][...])
            p1_ref[f] = p1
            p2_ref[f] = p2

    start = ib * BI
    outs = (o0, o1, o2, o3, o4)
    for f in range(1 + NSC):
        E = e_ref[...] if f == 0 else embs_ref[f - 1]
        e_blk = (e_ref[pl.ds(start, BI), :] if f == 0
                 else embs_ref[f - 1, pl.ds(start, BI), :])
        outs[f][...] = _pair_block(E, e_blk, Ws[f][..., 2 * FEAT:],
                                   p1_ref[f, pl.ds(start, BI), :], p2_ref[f])


@jax.jit
def _encode(ids, tab, embs, W0, W1, W2, W3r, W4, b0, b1, b2, b3, b4):
    return pl.pallas_call(
        _tc_body,
        grid=(GRID,),
        in_specs=(
            [pl.BlockSpec((N, 1), lambda i: (0, 0)),
             pl.BlockSpec((TVOC, FEAT), lambda i: (0, 0)),
             pl.BlockSpec((NSC, N, FEAT), lambda i: (0, 0, 0))]
            + [pl.BlockSpec((LATENT, 3 * FEAT), lambda i: (0, 0))] * (1 + NSC)
            + [pl.BlockSpec((1, LATENT), lambda i: (0, 0))] * (1 + NSC)
        ),
        out_specs=[pl.BlockSpec((BI * N, LATENT), lambda i: (i, 0))] * (1 + NSC),
        out_shape=[jax.ShapeDtypeStruct((N * N, LATENT), jnp.float32)] * (1 + NSC),
        scratch_shapes=[pltpu.VMEM((N, FEAT), jnp.float32),
                        pltpu.VMEM((1 + NSC, N, LATENT), jnp.float32),
                        pltpu.VMEM((1 + NSC, N, LATENT), jnp.float32)],
    )(ids, tab, embs, W0, W1, W2, W3r, W4, b0, b1, b2, b3, b4)


def kernel(typ_ids, typ_table, typ_W, typ_b, pol_ids, pol_table, pol_W, pol_b,
           mod_ids, mod_table, mod_W, mod_b, gen_ids, gen_table, gen_W, gen_b,
           ten_ids, ten_table, ten_W, ten_b):
    ids = tuple(i.astype(jnp.int32)
                for i in (pol_ids, mod_ids, gen_ids, ten_ids))
    embs = _sc_gather(pol_table, ids[0], mod_table, ids[1],
                      gen_table, ids[2], ten_table, ids[3])
    outs = _encode(typ_ids.astype(jnp.int32).reshape(N, 1), typ_table, embs,
                   typ_W, pol_W, mod_W, gen_W, ten_W,
                   typ_b.reshape(1, LATENT), pol_b.reshape(1, LATENT),
                   mod_b.reshape(1, LATENT), gen_b.reshape(1, LATENT),
                   ten_b.reshape(1, LATENT))
    return tuple(outs)


# speedup vs baseline: 1.0451x; 1.0010x over previous
import functools

import jax
import jax.numpy as jnp
from jax import lax
from jax.experimental import pallas as pl
from jax.experimental.pallas import tpu as pltpu
from jax.experimental.pallas import tpu_sc as plsc

N = 256
FEAT = 128
LATENT = 256
TVOC = 33
NSC = 4
BI = 8
GRID = N // BI

SC_NC = 2
SC_NS = 16
NW = SC_NC * SC_NS
BPW = N // NW

_DN = (((1,), (1,)), ((), ()))


@functools.partial(
    pl.kernel,
    mesh=plsc.VectorSubcoreMesh(core_axis_name="c", subcore_axis_name="s"),
    out_type=jax.ShapeDtypeStruct((NSC, N, FEAT), jnp.float32),
    scratch_types=(
        [pltpu.VMEM((BPW,), jnp.int32)] * NSC
        + [pltpu.VMEM((BPW, FEAT), jnp.float32)] * NSC
        + [pltpu.SemaphoreType.DMA] * NSC
    ),
)
def _sc_gather(t0, i0, t1, i1, t2, i2, t3, i3, out_hbm,
               x0, x1, x2, x3, r0, r1, r2, r3, s0, s1, s2, s3):
    wid = lax.axis_index("s") * SC_NC + lax.axis_index("c")
    base = wid * BPW
    tabs = (t0, t1, t2, t3)
    ids = (i0, i1, i2, i3)
    idx = (x0, x1, x2, x3)
    rows = (r0, r1, r2, r3)
    sems = (s0, s1, s2, s3)
    idx_c = [pltpu.async_copy(ids[f].at[pl.ds(base, BPW)], idx[f], sems[f])
             for f in range(NSC)]
    gat_c = []
    for f in range(NSC):
        idx_c[f].wait()
        gat_c.append(pltpu.async_copy(tabs[f].at[idx[f]], rows[f], sems[f]))
    out_c = []
    for f in range(NSC):
        gat_c[f].wait()
        out_c.append(pltpu.async_copy(rows[f], out_hbm.at[f, pl.ds(base, BPW)],
                                      sems[f]))
    for c in out_c:
        c.wait()


def _pair_block(E, e_blk, W3, p1_blk, p2b):
    R = e_blk[:, None, :] * E[None, :, :]
    M = lax.dot_general(R, W3, (((2,), (1,)), ((), ())),
                        preferred_element_type=jnp.float32)
    out3 = jnp.maximum(M + p1_blk[:, None, :] + p2b[None, :, :], 0.0)
    return out3.reshape(BI * N, LATENT)


def _p_pair(E, W, b):
    p1 = lax.dot_general(E, W[:, :FEAT], _DN,
                         preferred_element_type=jnp.float32)
    p2 = (lax.dot_general(E, W[:, FEAT:2 * FEAT], _DN,
                          preferred_element_type=jnp.float32) + b)
    return p1, p2


def _tc_body(ids_ref, tab_ref, embs_ref, W0, W1, W2, W3r, W4,
             b0, b1, b2, b3, b4, o0, o1, o2, o3, o4,
             e_ref, p1_ref, p2_ref):
    ib = pl.program_id(0)
    Ws = (W0, W1, W2, W3r, W4)
    bs = (b0, b1, b2, b3, b4)

    @pl.when(ib == 0)
    def _():
        onehot = (ids_ref[...] == lax.broadcasted_iota(jnp.int32, (N, TVOC), 1)
                  ).astype(jnp.float32)
        e_ref[...] = lax.dot_general(
            onehot, tab_ref[...], (((1,), (0,)), ((), ())),
            precision=lax.Precision.HIGHEST,
            preferred_element_type=jnp.float32)
        for f in range(1 + NSC):
            E = e_ref[...] if f == 0 else embs_ref[f - 1]
            p1, p2 = _p_pair(E, Ws[f][...], bs[f][...])
            p1_ref[f] = p1
            p2_ref[f] = p2

    start = ib * BI
    outs = (o0, o1, o2, o3, o4)
    for f in range(1 + NSC):
        E = e_ref[...] if f == 0 else embs_ref[f - 1]
        e_blk = (e_ref[pl.ds(start, BI), :] if f == 0
                 else embs_ref[f - 1, pl.ds(start, BI), :])
        outs[f][...] = _pair_block(E, e_blk, Ws[f][..., 2 * FEAT:],
                                   p1_ref[f, pl.ds(start, BI), :], p2_ref[f])


@jax.jit
def _encode(ids, tab, embs, W0, W1, W2, W3r, W4, b0, b1, b2, b3, b4):
    return pl.pallas_call(
        _tc_body,
        grid=(GRID,),
        in_specs=(
            [pl.BlockSpec((N, 1), lambda i: (0, 0)),
             pl.BlockSpec((TVOC, FEAT), lambda i: (0, 0)),
             pl.BlockSpec((NSC, N, FEAT), lambda i: (0, 0, 0))]
            + [pl.BlockSpec((LATENT, 3 * FEAT), lambda i: (0, 0))] * (1 + NSC)
            + [pl.BlockSpec((1, LATENT), lambda i: (0, 0))] * (1 + NSC)
        ),
        out_specs=[pl.BlockSpec((BI * N, LATENT), lambda i: (i, 0))] * (1 + NSC),
        out_shape=[jax.ShapeDtypeStruct((N * N, LATENT), jnp.float32)] * (1 + NSC),
        scratch_shapes=[pltpu.VMEM((N, FEAT), jnp.float32),
                        pltpu.VMEM((1 + NSC, N, LATENT), jnp.float32),
                        pltpu.VMEM((1 + NSC, N, LATENT), jnp.float32)],
    )(ids, tab, embs, W0, W1, W2, W3r, W4, b0, b1, b2, b3, b4)


def kernel(typ_ids, typ_table, typ_W, typ_b, pol_ids, pol_table, pol_W, pol_b,
           mod_ids, mod_table, mod_W, mod_b, gen_ids, gen_table, gen_W, gen_b,
           ten_ids, ten_table, ten_W, ten_b):
    ids = tuple(i.astype(jnp.int32)
                for i in (pol_ids, mod_ids, gen_ids, ten_ids))
    embs = _sc_gather(pol_table, ids[0], mod_table, ids[1],
                      gen_table, ids[2], ten_table, ids[3])
    outs = _encode(typ_ids.astype(jnp.int32).reshape(N, 1), typ_table, embs,
                   typ_W, pol_W, mod_W, gen_W, ten_W,
                   typ_b.reshape(1, LATENT), pol_b.reshape(1, LATENT),
                   mod_b.reshape(1, LATENT), gen_b.reshape(1, LATENT),
                   ten_b.reshape(1, LATENT))
    return tuple(outs)
